# baseline (speedup 1.0000x reference)
import functools

import jax
import jax.numpy as jnp
from jax.experimental import pallas as pl
from jax.experimental.pallas import tpu as pltpu


def _plane_sum_kernel(x_ref, o_ref, *, inv_hw, hw_steps):
    k = pl.program_id(1)
    s = jnp.sum(x_ref[...], axis=0)

    @pl.when(k == 0)
    def _():
        o_ref[...] = s

    @pl.when(k > 0)
    def _():
        o_ref[...] += s

    @pl.when(k == hw_steps - 1)
    def _():
        o_ref[...] *= inv_hw


def _global_avg_pool2d(x_nchw, *, n_blk=16, hw_blk=7):
    N, C, H, W = x_nchw.shape
    HW = H * W
    hw_steps = HW // hw_blk

    planes = jnp.transpose(x_nchw, (2, 3, 0, 1)).reshape(HW, N, C)
    inv_hw = 1.0 / float(HW)

    out2d = pl.pallas_call(
        functools.partial(_plane_sum_kernel, inv_hw=inv_hw,
                          hw_steps=hw_steps),
        out_shape=jax.ShapeDtypeStruct((N, C), x_nchw.dtype),
        grid_spec=pltpu.PrefetchScalarGridSpec(
            num_scalar_prefetch=0,
            grid=(N // n_blk, hw_steps),
            in_specs=[pl.BlockSpec((hw_blk, n_blk, C),
                                   lambda i, k: (k, i, 0))],
            out_specs=pl.BlockSpec((n_blk, C), lambda i, k: (i, 0)),
        ),
        compiler_params=pltpu.CompilerParams(
            dimension_semantics=("parallel", "arbitrary")),
    )(planes)

    return out2d.reshape(N, C, 1, 1)


def kernel(x_nchw):
    return _global_avg_pool2d(x_nchw)


# speedup vs baseline: 2.1825x; 2.1825x over previous
import functools

import jax
import jax.numpy as jnp
from jax.experimental import pallas as pl
from jax.experimental.pallas import tpu as pltpu


def _plane_sum_kernel(x_ref, o_ref, *, inv_hw):
    s = jnp.sum(x_ref[...], axis=0)
    o_ref[...] = (s * inv_hw).astype(o_ref.dtype)


def _global_avg_pool2d(x_nchw, *, n_blk=16):
    N, C, H, W = x_nchw.shape
    HW = H * W

    planes = jnp.transpose(x_nchw, (2, 3, 0, 1)).reshape(HW, N, C)
    inv_hw = 1.0 / float(HW)

    out2d = pl.pallas_call(
        functools.partial(_plane_sum_kernel, inv_hw=inv_hw),
        out_shape=jax.ShapeDtypeStruct((N, C), x_nchw.dtype),
        grid_spec=pltpu.PrefetchScalarGridSpec(
            num_scalar_prefetch=0,
            grid=(N // n_blk,),
            in_specs=[pl.BlockSpec((HW, n_blk, C), lambda i: (0, i, 0))],
            out_specs=pl.BlockSpec((n_blk, C), lambda i: (i, 0)),
        ),
        compiler_params=pltpu.CompilerParams(
            dimension_semantics=("parallel",)),
    )(planes)

    return out2d.reshape(N, C, 1, 1)


def kernel(x_nchw):
    return _global_avg_pool2d(x_nchw)
